# baseline (speedup 1.0000x reference)
import functools

import jax
import jax.numpy as jnp
from jax import lax
from jax.experimental import pallas as pl
from jax.experimental.pallas import tpu as pltpu
from jax.experimental.pallas import tpu_sc as plsc

_NC = 2
_NS = 16
_NW = _NC * _NS
_BLK = 25000
_SC_ROWS = 50000
_CHUNK = 312


def _tc_copy_head(ent_ref, _sc_ref, ent_out):
    ent_out[...] = ent_ref[...]


def _sc_part(entity_emb, rel_emb):
    n_ent, d_ent = entity_emb.shape
    start = n_ent - _SC_ROWS
    rows = (_SC_ROWS // _NW) // 8 * 8
    nchunks = rows // _CHUNK
    rem = rows - nchunks * _CHUNK
    tail_base = start + rows * _NW
    tail = n_ent - tail_base

    mesh = plsc.VectorSubcoreMesh(core_axis_name="c", subcore_axis_name="s")

    @functools.partial(
        pl.kernel,
        mesh=mesh,
        out_type=[
            jax.ShapeDtypeStruct(entity_emb.shape, entity_emb.dtype),
            jax.ShapeDtypeStruct(rel_emb.shape, rel_emb.dtype),
        ],
        scratch_types=[
            pltpu.VMEM((_CHUNK, d_ent), entity_emb.dtype),
            pltpu.VMEM((_CHUNK, d_ent), entity_emb.dtype),
            pltpu.SemaphoreType.DMA,
            pltpu.SemaphoreType.DMA,
            pltpu.SemaphoreType.DMA,
            pltpu.SemaphoreType.DMA,
            pltpu.SemaphoreType.DMA,
        ],
    )
    def _body(ent_hbm, rel_hbm, ent_out, rel_out,
              buf0, buf1, isem0, isem1, osem0, osem1, rsem):
        wid = lax.axis_index("s") * _NC + lax.axis_index("c")
        base = start + wid * rows
        bufs = (buf0, buf1)
        isems = (isem0, isem1)
        osems = (osem0, osem1)

        @pl.when(wid == 0)
        def _():
            pltpu.make_async_copy(rel_hbm, rel_out, rsem).start()
            if tail:
                pltpu.make_async_copy(
                    ent_hbm.at[pl.ds(tail_base, tail)],
                    ent_out.at[pl.ds(tail_base, tail)],
                    rsem,
                ).start()

        out_cps = [None, None]
        chunks = [_CHUNK] * nchunks + ([rem] if rem else [])
        lo = base
        for i, c in enumerate(chunks):
            b = i % 2
            if out_cps[b] is not None:
                out_cps[b].wait()
            icp = pltpu.make_async_copy(
                ent_hbm.at[pl.ds(lo, c)], bufs[b].at[pl.ds(0, c)], isems[b]
            )
            icp.start()
            icp.wait()
            ocp = pltpu.make_async_copy(
                bufs[b].at[pl.ds(0, c)], ent_out.at[pl.ds(lo, c)], osems[b]
            )
            ocp.start()
            out_cps[b] = ocp
            lo += c
        for cp in out_cps:
            if cp is not None:
                cp.wait()

        @pl.when(wid == 0)
        def _():
            if tail:
                pltpu.make_async_copy(
                    ent_hbm.at[pl.ds(tail_base, tail)],
                    ent_out.at[pl.ds(tail_base, tail)],
                    rsem,
                ).wait()
            pltpu.make_async_copy(rel_hbm, rel_out, rsem).wait()

    return _body(entity_emb, rel_emb)


def kernel(x_dict, edge_index, entity_emb, rel_emb):
    del x_dict, edge_index
    n_ent, d_ent = entity_emb.shape
    head = n_ent - _SC_ROWS

    sc_ent, rel = _sc_part(entity_emb, rel_emb)

    ent = pl.pallas_call(
        _tc_copy_head,
        grid=(head // _BLK,),
        in_specs=[
            pl.BlockSpec((_BLK, d_ent), lambda i: (i, 0)),
            pl.BlockSpec(memory_space=pltpu.MemorySpace.HBM),
        ],
        out_specs=pl.BlockSpec((_BLK, d_ent), lambda i: (i, 0)),
        out_shape=jax.ShapeDtypeStruct((n_ent, d_ent), entity_emb.dtype),
        input_output_aliases={1: 0},
    )(entity_emb, sc_ent)
    return (ent, rel)


# speedup vs baseline: 1.0039x; 1.0039x over previous
import functools

import jax
import jax.numpy as jnp
from jax import lax
from jax.experimental import pallas as pl
from jax.experimental.pallas import tpu as pltpu
from jax.experimental.pallas import tpu_sc as plsc

_NC = 2
_NS = 16
_NW = _NC * _NS
_BLK = 25000
_SC_ROWS = 25000
_CHUNK = 312


def _tc_copy_head(ent_ref, _sc_ref, ent_out):
    ent_out[...] = ent_ref[...]


def _sc_part(entity_emb, rel_emb):
    n_ent, d_ent = entity_emb.shape
    start = n_ent - _SC_ROWS
    rows = (_SC_ROWS // _NW) // 8 * 8
    nchunks = rows // _CHUNK
    rem = rows - nchunks * _CHUNK
    tail_base = start + rows * _NW
    tail = n_ent - tail_base

    mesh = plsc.VectorSubcoreMesh(core_axis_name="c", subcore_axis_name="s")

    @functools.partial(
        pl.kernel,
        mesh=mesh,
        out_type=[
            jax.ShapeDtypeStruct(entity_emb.shape, entity_emb.dtype),
            jax.ShapeDtypeStruct(rel_emb.shape, rel_emb.dtype),
        ],
        scratch_types=[
            pltpu.VMEM((_CHUNK, d_ent), entity_emb.dtype),
            pltpu.VMEM((_CHUNK, d_ent), entity_emb.dtype),
            pltpu.SemaphoreType.DMA,
            pltpu.SemaphoreType.DMA,
            pltpu.SemaphoreType.DMA,
            pltpu.SemaphoreType.DMA,
            pltpu.SemaphoreType.DMA,
        ],
    )
    def _body(ent_hbm, rel_hbm, ent_out, rel_out,
              buf0, buf1, isem0, isem1, osem0, osem1, rsem):
        wid = lax.axis_index("s") * _NC + lax.axis_index("c")
        base = start + wid * rows
        bufs = (buf0, buf1)
        isems = (isem0, isem1)
        osems = (osem0, osem1)

        @pl.when(wid == 0)
        def _():
            pltpu.make_async_copy(rel_hbm, rel_out, rsem).start()
            if tail:
                pltpu.make_async_copy(
                    ent_hbm.at[pl.ds(tail_base, tail)],
                    ent_out.at[pl.ds(tail_base, tail)],
                    rsem,
                ).start()

        out_cps = [None, None]
        chunks = [_CHUNK] * nchunks + ([rem] if rem else [])
        lo = base
        for i, c in enumerate(chunks):
            b = i % 2
            if out_cps[b] is not None:
                out_cps[b].wait()
            icp = pltpu.make_async_copy(
                ent_hbm.at[pl.ds(lo, c)], bufs[b].at[pl.ds(0, c)], isems[b]
            )
            icp.start()
            icp.wait()
            ocp = pltpu.make_async_copy(
                bufs[b].at[pl.ds(0, c)], ent_out.at[pl.ds(lo, c)], osems[b]
            )
            ocp.start()
            out_cps[b] = ocp
            lo += c
        for cp in out_cps:
            if cp is not None:
                cp.wait()

        @pl.when(wid == 0)
        def _():
            if tail:
                pltpu.make_async_copy(
                    ent_hbm.at[pl.ds(tail_base, tail)],
                    ent_out.at[pl.ds(tail_base, tail)],
                    rsem,
                ).wait()
            pltpu.make_async_copy(rel_hbm, rel_out, rsem).wait()

    return _body(entity_emb, rel_emb)


def kernel(x_dict, edge_index, entity_emb, rel_emb):
    del x_dict, edge_index
    n_ent, d_ent = entity_emb.shape
    head = n_ent - _SC_ROWS

    sc_ent, rel = _sc_part(entity_emb, rel_emb)

    ent = pl.pallas_call(
        _tc_copy_head,
        grid=(head // _BLK,),
        in_specs=[
            pl.BlockSpec((_BLK, d_ent), lambda i: (i, 0)),
            pl.BlockSpec(memory_space=pltpu.MemorySpace.HBM),
        ],
        out_specs=pl.BlockSpec((_BLK, d_ent), lambda i: (i, 0)),
        out_shape=jax.ShapeDtypeStruct((n_ent, d_ent), entity_emb.dtype),
        input_output_aliases={1: 0},
    )(entity_emb, sc_ent)
    return (ent, rel)
